# Initial kernel scaffold; baseline (speedup 1.0000x reference)
import functools

import jax
import jax.numpy as jnp
from jax import lax
from jax.experimental import pallas as pl
from jax.experimental.pallas import tpu as pltpu
from jax.experimental.pallas import tpu_sc as plsc

N_FIELDS = 26
VOCAB = 100000
EMB_DIM = 16
N_CONT = 13
B = 16384
EMB_COLS = N_FIELDS * EMB_DIM
EPS = 1e-5

CH = 128
BLK = 512
H1P = 1024
H2P = 512
CONTP = 16
OUTP = 128


def _sc_gather(table2d, fidx):
    info = plsc.get_sparse_core_info()
    nc, ns = info.num_cores, info.num_subcores
    nw = nc * ns
    total = B * N_FIELDS
    per_w = total // nw
    nch = per_w // CH
    assert total == nw * nch * CH
    fidx3 = fidx.reshape(nw, nch, CH)
    mesh = plsc.VectorSubcoreMesh(core_axis_name="c", subcore_axis_name="s")

    @functools.partial(
        pl.kernel,
        mesh=mesh,
        out_type=jax.ShapeDtypeStruct((nw, nch, CH, EMB_DIM), jnp.float32),
        scratch_types=[
            pltpu.VMEM((nch, CH), jnp.int32),
            pltpu.VMEM((CH, EMB_DIM), jnp.float32),
            pltpu.VMEM((CH, EMB_DIM), jnp.float32),
            pltpu.SemaphoreType.DMA,
            pltpu.SemaphoreType.DMA,
        ],
    )
    def gk(table_hbm, fidx_hbm, out_hbm, idx_v, buf0, buf1, sem0, sem1):
        wid = lax.axis_index("s") * nc + lax.axis_index("c")
        pltpu.sync_copy(fidx_hbm.at[wid], idx_v)

        def body(c2, carry):
            c0 = 2 * c2
            ga = pltpu.async_copy(table_hbm.at[idx_v.at[c0]], buf0, sem0)
            gb = pltpu.async_copy(table_hbm.at[idx_v.at[c0 + 1]], buf1, sem1)
            ga.wait()
            pltpu.sync_copy(buf0, out_hbm.at[wid, c0])
            gb.wait()
            pltpu.sync_copy(buf1, out_hbm.at[wid, c0 + 1])
            return carry

        lax.fori_loop(0, nch // 2, body, 0)

    return gk(table2d, fidx3)


def _tc1(xemb, xc, w1e, w1c, b1, bng, bnb):
    nblk = B // BLK

    def body(xe_ref, xc_ref, w1e_ref, w1c_ref, b1_ref, g_ref, bb_ref,
             h1_ref, st_ref, aff_ref):
        i = pl.program_id(0)

        @pl.when(i == 0)
        def _():
            xcf = xc_ref[...]
            m = jnp.mean(xcf, axis=0, keepdims=True)
            v = jnp.mean(xcf * xcf, axis=0, keepdims=True) - m * m
            sc = g_ref[...] / jnp.sqrt(v + EPS)
            aff_ref[0:1, :] = sc
            aff_ref[1:2, :] = bb_ref[...] - m * sc
            st_ref[...] = jnp.zeros_like(st_ref)

        xcb = xc_ref[pl.ds(i * BLK, BLK), :]
        xcn = xcb * aff_ref[0:1, :] + aff_ref[1:2, :]
        h = jnp.dot(xe_ref[...], w1e_ref[...], preferred_element_type=jnp.float32)
        h = h + jnp.dot(xcn, w1c_ref[...], preferred_element_type=jnp.float32)
        h = jnp.maximum(h + b1_ref[...], 0.0)
        h1_ref[...] = h
        st_ref[0:1, :] += jnp.sum(h, axis=0, keepdims=True)
        st_ref[1:2, :] += jnp.sum(h * h, axis=0, keepdims=True)

    return pl.pallas_call(
        body,
        grid=(nblk,),
        in_specs=[
            pl.BlockSpec((BLK, EMB_COLS), lambda i: (i, 0)),
            pl.BlockSpec((B, CONTP), lambda i: (0, 0)),
            pl.BlockSpec((EMB_COLS, H1P), lambda i: (0, 0)),
            pl.BlockSpec((CONTP, H1P), lambda i: (0, 0)),
            pl.BlockSpec((1, H1P), lambda i: (0, 0)),
            pl.BlockSpec((1, CONTP), lambda i: (0, 0)),
            pl.BlockSpec((1, CONTP), lambda i: (0, 0)),
        ],
        out_specs=[
            pl.BlockSpec((BLK, H1P), lambda i: (i, 0)),
            pl.BlockSpec((8, H1P), lambda i: (0, 0)),
        ],
        out_shape=[
            jax.ShapeDtypeStruct((B, H1P), jnp.float32),
            jax.ShapeDtypeStruct((8, H1P), jnp.float32),
        ],
        scratch_shapes=[pltpu.VMEM((8, CONTP), jnp.float32)],
        compiler_params=pltpu.CompilerParams(
            dimension_semantics=("arbitrary",)),
    )(xemb, xc, w1e, w1c, b1, bng, bnb)


def _tc_mid(h1, st1, w2, b2, g1, bb1):
    nblk = B // BLK

    def body(h1_ref, st_ref_in, w2_ref, b2_ref, g_ref, bb_ref,
             h2_ref, st2_ref, aff_ref):
        i = pl.program_id(0)

        @pl.when(i == 0)
        def _():
            m = st_ref_in[0:1, :] * (1.0 / B)
            v = st_ref_in[1:2, :] * (1.0 / B) - m * m
            sc = g_ref[...] / jnp.sqrt(v + EPS)
            aff_ref[0:1, :] = sc
            aff_ref[1:2, :] = bb_ref[...] - m * sc
            st2_ref[...] = jnp.zeros_like(st2_ref)

        h1n = h1_ref[...] * aff_ref[0:1, :] + aff_ref[1:2, :]
        h = jnp.dot(h1n, w2_ref[...], preferred_element_type=jnp.float32)
        h = jnp.maximum(h + b2_ref[...], 0.0)
        h2_ref[...] = h
        st2_ref[0:1, :] += jnp.sum(h, axis=0, keepdims=True)
        st2_ref[1:2, :] += jnp.sum(h * h, axis=0, keepdims=True)

    return pl.pallas_call(
        body,
        grid=(nblk,),
        in_specs=[
            pl.BlockSpec((BLK, H1P), lambda i: (i, 0)),
            pl.BlockSpec((8, H1P), lambda i: (0, 0)),
            pl.BlockSpec((H1P, H2P), lambda i: (0, 0)),
            pl.BlockSpec((1, H2P), lambda i: (0, 0)),
            pl.BlockSpec((1, H1P), lambda i: (0, 0)),
            pl.BlockSpec((1, H1P), lambda i: (0, 0)),
        ],
        out_specs=[
            pl.BlockSpec((BLK, H2P), lambda i: (i, 0)),
            pl.BlockSpec((8, H2P), lambda i: (0, 0)),
        ],
        out_shape=[
            jax.ShapeDtypeStruct((B, H2P), jnp.float32),
            jax.ShapeDtypeStruct((8, H2P), jnp.float32),
        ],
        scratch_shapes=[pltpu.VMEM((8, H1P), jnp.float32)],
        compiler_params=pltpu.CompilerParams(
            dimension_semantics=("arbitrary",)),
    )(h1, st1, w2, b2, g1, bb1)


def _tc_last(h2, st2, w3, b3, g2, bb2):
    nblk = B // BLK

    def body(h2_ref, st_ref_in, w3_ref, b3_ref, g_ref, bb_ref,
             o_ref, aff_ref):
        i = pl.program_id(0)

        @pl.when(i == 0)
        def _():
            m = st_ref_in[0:1, :] * (1.0 / B)
            v = st_ref_in[1:2, :] * (1.0 / B) - m * m
            sc = g_ref[...] / jnp.sqrt(v + EPS)
            aff_ref[0:1, :] = sc
            aff_ref[1:2, :] = bb_ref[...] - m * sc

        h2n = h2_ref[...] * aff_ref[0:1, :] + aff_ref[1:2, :]
        o = jnp.dot(h2n, w3_ref[...], preferred_element_type=jnp.float32)
        o_ref[...] = o + b3_ref[...]

    return pl.pallas_call(
        body,
        grid=(nblk,),
        in_specs=[
            pl.BlockSpec((BLK, H2P), lambda i: (i, 0)),
            pl.BlockSpec((8, H2P), lambda i: (0, 0)),
            pl.BlockSpec((H2P, OUTP), lambda i: (0, 0)),
            pl.BlockSpec((1, OUTP), lambda i: (0, 0)),
            pl.BlockSpec((1, H2P), lambda i: (0, 0)),
            pl.BlockSpec((1, H2P), lambda i: (0, 0)),
        ],
        out_specs=pl.BlockSpec((BLK, OUTP), lambda i: (i, 0)),
        out_shape=jax.ShapeDtypeStruct((B, OUTP), jnp.float32),
        scratch_shapes=[pltpu.VMEM((8, H2P), jnp.float32)],
        compiler_params=pltpu.CompilerParams(
            dimension_semantics=("arbitrary",)),
    )(h2, st2, w3, b3, g2, bb2)


def kernel(x_cat, x_cont, emb_tables, bn_cont_g, bn_cont_b,
           W1, b1, bn1_g, bn1_b, W2, b2, bn2_g, bn2_b, W3, b3):
    offs = (jnp.arange(N_FIELDS, dtype=jnp.int32) * VOCAB)[None, :]
    fidx = (x_cat.astype(jnp.int32) + offs).reshape(-1)
    table2d = emb_tables.reshape(N_FIELDS * VOCAB, EMB_DIM)

    xc = jnp.pad(x_cont, ((0, 0), (0, CONTP - N_CONT)))
    bncg = jnp.pad(bn_cont_g, (0, CONTP - N_CONT)).reshape(1, CONTP)
    bncb = jnp.pad(bn_cont_b, (0, CONTP - N_CONT)).reshape(1, CONTP)

    n1 = W1.shape[0]
    n2 = W2.shape[0]
    w1e = jnp.pad(W1[:, :EMB_COLS].T, ((0, 0), (0, H1P - n1)))
    w1c = jnp.pad(W1[:, EMB_COLS:].T,
                  ((0, CONTP - N_CONT), (0, H1P - n1)))
    b1p = jnp.pad(b1, (0, H1P - n1)).reshape(1, H1P)
    g1p = jnp.pad(bn1_g, (0, H1P - n1)).reshape(1, H1P)
    bb1p = jnp.pad(bn1_b, (0, H1P - n1)).reshape(1, H1P)

    w2p = jnp.pad(W2.T, ((0, H1P - n1), (0, H2P - n2)))
    b2p = jnp.pad(b2, (0, H2P - n2)).reshape(1, H2P)
    g2p = jnp.pad(bn2_g, (0, H2P - n2)).reshape(1, H2P)
    bb2p = jnp.pad(bn2_b, (0, H2P - n2)).reshape(1, H2P)

    w3p = jnp.pad(W3.T, ((0, H2P - n2), (0, OUTP - 1)))
    b3p = jnp.pad(b3, (0, OUTP - 1)).reshape(1, OUTP)

    xemb = _sc_gather(table2d, fidx).reshape(B, EMB_COLS)
    h1, st1 = _tc1(xemb, xc, w1e, w1c, b1p, bncg, bncb)
    h2, st2 = _tc_mid(h1, st1, w2p, b2p, g1p, bb1p)
    o = _tc_last(h2, st2, w3p, b3p, g2p, bb2p)
    return o[:, :1]


# speedup vs baseline: 7.0322x; 7.0322x over previous
import functools

import jax
import jax.numpy as jnp
from jax import lax
from jax.experimental import pallas as pl
from jax.experimental.pallas import tpu as pltpu
from jax.experimental.pallas import tpu_sc as plsc

N_FIELDS = 26
VOCAB = 100000
EMB_DIM = 16
N_CONT = 13
B = 16384
EMB_COLS = N_FIELDS * EMB_DIM
EPS = 1e-5

CH = 128
BLK = 512
H1P = 1024
H2P = 512
CONTP = 16
OUTP = 128


def _sc_gather(table2d, fidx):
    info = plsc.get_sparse_core_info()
    nc, ns = info.num_cores, info.num_subcores
    nw = nc * ns
    total = B * N_FIELDS
    per_w = total // nw
    nch = per_w // CH
    assert total == nw * nch * CH
    fidx3 = fidx.reshape(nw, nch, CH)
    mesh = plsc.VectorSubcoreMesh(core_axis_name="c", subcore_axis_name="s")

    @functools.partial(
        pl.kernel,
        mesh=mesh,
        out_type=jax.ShapeDtypeStruct((nw, nch, CH, EMB_DIM), jnp.float32),
        scratch_types=[
            pltpu.VMEM((nch, CH), jnp.int32),
            pltpu.VMEM((CH, EMB_DIM), jnp.float32),
            pltpu.VMEM((CH, EMB_DIM), jnp.float32),
            pltpu.SemaphoreType.DMA,
            pltpu.SemaphoreType.DMA,
        ],
        compiler_params=pltpu.CompilerParams(use_tc_tiling_on_sc=False),
    )
    def gk(table_hbm, fidx_hbm, out_hbm, idx_v, buf0, buf1, sem0, sem1):
        wid = lax.axis_index("s") * nc + lax.axis_index("c")
        pltpu.sync_copy(fidx_hbm.at[wid], idx_v)

        def body(c2, carry):
            c0 = 2 * c2
            ga = pltpu.async_copy(table_hbm.at[idx_v.at[c0]], buf0, sem0)
            gb = pltpu.async_copy(table_hbm.at[idx_v.at[c0 + 1]], buf1, sem1)
            ga.wait()
            pltpu.sync_copy(buf0, out_hbm.at[wid, c0])
            gb.wait()
            pltpu.sync_copy(buf1, out_hbm.at[wid, c0 + 1])
            return carry

        lax.fori_loop(0, nch // 2, body, 0)

    return gk(table2d, fidx3)


def _tc1(xemb, xc, w1e, w1c, b1, bng, bnb):
    nblk = B // BLK

    def body(xe_ref, xc_ref, w1e_ref, w1c_ref, b1_ref, g_ref, bb_ref,
             h1_ref, st_ref, aff_ref):
        i = pl.program_id(0)

        @pl.when(i == 0)
        def _():
            xcf = xc_ref[...]
            m = jnp.mean(xcf, axis=0, keepdims=True)
            v = jnp.mean(xcf * xcf, axis=0, keepdims=True) - m * m
            sc = g_ref[...] / jnp.sqrt(v + EPS)
            aff_ref[0:1, :] = sc
            aff_ref[1:2, :] = bb_ref[...] - m * sc
            st_ref[...] = jnp.zeros_like(st_ref)

        xcb = xc_ref[pl.ds(i * BLK, BLK), :]
        xcn = xcb * aff_ref[0:1, :] + aff_ref[1:2, :]
        h = jnp.dot(xe_ref[...], w1e_ref[...], preferred_element_type=jnp.float32)
        h = h + jnp.dot(xcn, w1c_ref[...], preferred_element_type=jnp.float32)
        h = jnp.maximum(h + b1_ref[...], 0.0)
        h1_ref[...] = h
        st_ref[0:1, :] += jnp.sum(h, axis=0, keepdims=True)
        st_ref[1:2, :] += jnp.sum(h * h, axis=0, keepdims=True)

    return pl.pallas_call(
        body,
        grid=(nblk,),
        in_specs=[
            pl.BlockSpec((BLK, EMB_COLS), lambda i: (i, 0)),
            pl.BlockSpec((B, CONTP), lambda i: (0, 0)),
            pl.BlockSpec((EMB_COLS, H1P), lambda i: (0, 0)),
            pl.BlockSpec((CONTP, H1P), lambda i: (0, 0)),
            pl.BlockSpec((1, H1P), lambda i: (0, 0)),
            pl.BlockSpec((1, CONTP), lambda i: (0, 0)),
            pl.BlockSpec((1, CONTP), lambda i: (0, 0)),
        ],
        out_specs=[
            pl.BlockSpec((BLK, H1P), lambda i: (i, 0)),
            pl.BlockSpec((8, H1P), lambda i: (0, 0)),
        ],
        out_shape=[
            jax.ShapeDtypeStruct((B, H1P), jnp.float32),
            jax.ShapeDtypeStruct((8, H1P), jnp.float32),
        ],
        scratch_shapes=[pltpu.VMEM((8, CONTP), jnp.float32)],
        compiler_params=pltpu.CompilerParams(
            dimension_semantics=("arbitrary",)),
    )(xemb, xc, w1e, w1c, b1, bng, bnb)


def _tc_mid(h1, st1, w2, b2, g1, bb1):
    nblk = B // BLK

    def body(h1_ref, st_ref_in, w2_ref, b2_ref, g_ref, bb_ref,
             h2_ref, st2_ref, aff_ref):
        i = pl.program_id(0)

        @pl.when(i == 0)
        def _():
            m = st_ref_in[0:1, :] * (1.0 / B)
            v = st_ref_in[1:2, :] * (1.0 / B) - m * m
            sc = g_ref[...] / jnp.sqrt(v + EPS)
            aff_ref[0:1, :] = sc
            aff_ref[1:2, :] = bb_ref[...] - m * sc
            st2_ref[...] = jnp.zeros_like(st2_ref)

        h1n = h1_ref[...] * aff_ref[0:1, :] + aff_ref[1:2, :]
        h = jnp.dot(h1n, w2_ref[...], preferred_element_type=jnp.float32)
        h = jnp.maximum(h + b2_ref[...], 0.0)
        h2_ref[...] = h
        st2_ref[0:1, :] += jnp.sum(h, axis=0, keepdims=True)
        st2_ref[1:2, :] += jnp.sum(h * h, axis=0, keepdims=True)

    return pl.pallas_call(
        body,
        grid=(nblk,),
        in_specs=[
            pl.BlockSpec((BLK, H1P), lambda i: (i, 0)),
            pl.BlockSpec((8, H1P), lambda i: (0, 0)),
            pl.BlockSpec((H1P, H2P), lambda i: (0, 0)),
            pl.BlockSpec((1, H2P), lambda i: (0, 0)),
            pl.BlockSpec((1, H1P), lambda i: (0, 0)),
            pl.BlockSpec((1, H1P), lambda i: (0, 0)),
        ],
        out_specs=[
            pl.BlockSpec((BLK, H2P), lambda i: (i, 0)),
            pl.BlockSpec((8, H2P), lambda i: (0, 0)),
        ],
        out_shape=[
            jax.ShapeDtypeStruct((B, H2P), jnp.float32),
            jax.ShapeDtypeStruct((8, H2P), jnp.float32),
        ],
        scratch_shapes=[pltpu.VMEM((8, H1P), jnp.float32)],
        compiler_params=pltpu.CompilerParams(
            dimension_semantics=("arbitrary",)),
    )(h1, st1, w2, b2, g1, bb1)


def _tc_last(h2, st2, w3, b3, g2, bb2):
    nblk = B // BLK

    def body(h2_ref, st_ref_in, w3_ref, b3_ref, g_ref, bb_ref,
             o_ref, aff_ref):
        i = pl.program_id(0)

        @pl.when(i == 0)
        def _():
            m = st_ref_in[0:1, :] * (1.0 / B)
            v = st_ref_in[1:2, :] * (1.0 / B) - m * m
            sc = g_ref[...] / jnp.sqrt(v + EPS)
            aff_ref[0:1, :] = sc
            aff_ref[1:2, :] = bb_ref[...] - m * sc

        h2n = h2_ref[...] * aff_ref[0:1, :] + aff_ref[1:2, :]
        o = jnp.dot(h2n, w3_ref[...], preferred_element_type=jnp.float32)
        o_ref[...] = o + b3_ref[...]

    return pl.pallas_call(
        body,
        grid=(nblk,),
        in_specs=[
            pl.BlockSpec((BLK, H2P), lambda i: (i, 0)),
            pl.BlockSpec((8, H2P), lambda i: (0, 0)),
            pl.BlockSpec((H2P, OUTP), lambda i: (0, 0)),
            pl.BlockSpec((1, OUTP), lambda i: (0, 0)),
            pl.BlockSpec((1, H2P), lambda i: (0, 0)),
            pl.BlockSpec((1, H2P), lambda i: (0, 0)),
        ],
        out_specs=pl.BlockSpec((BLK, OUTP), lambda i: (i, 0)),
        out_shape=jax.ShapeDtypeStruct((B, OUTP), jnp.float32),
        scratch_shapes=[pltpu.VMEM((8, H2P), jnp.float32)],
        compiler_params=pltpu.CompilerParams(
            dimension_semantics=("arbitrary",)),
    )(h2, st2, w3, b3, g2, bb2)


def kernel(x_cat, x_cont, emb_tables, bn_cont_g, bn_cont_b,
           W1, b1, bn1_g, bn1_b, W2, b2, bn2_g, bn2_b, W3, b3):
    offs = (jnp.arange(N_FIELDS, dtype=jnp.int32) * VOCAB)[None, :]
    fidx = (x_cat.astype(jnp.int32) + offs).reshape(-1)
    table2d = emb_tables.reshape(N_FIELDS * VOCAB, EMB_DIM)

    xc = jnp.pad(x_cont, ((0, 0), (0, CONTP - N_CONT)))
    bncg = jnp.pad(bn_cont_g, (0, CONTP - N_CONT)).reshape(1, CONTP)
    bncb = jnp.pad(bn_cont_b, (0, CONTP - N_CONT)).reshape(1, CONTP)

    n1 = W1.shape[0]
    n2 = W2.shape[0]
    w1e = jnp.pad(W1[:, :EMB_COLS].T, ((0, 0), (0, H1P - n1)))
    w1c = jnp.pad(W1[:, EMB_COLS:].T,
                  ((0, CONTP - N_CONT), (0, H1P - n1)))
    b1p = jnp.pad(b1, (0, H1P - n1)).reshape(1, H1P)
    g1p = jnp.pad(bn1_g, (0, H1P - n1)).reshape(1, H1P)
    bb1p = jnp.pad(bn1_b, (0, H1P - n1)).reshape(1, H1P)

    w2p = jnp.pad(W2.T, ((0, H1P - n1), (0, H2P - n2)))
    b2p = jnp.pad(b2, (0, H2P - n2)).reshape(1, H2P)
    g2p = jnp.pad(bn2_g, (0, H2P - n2)).reshape(1, H2P)
    bb2p = jnp.pad(bn2_b, (0, H2P - n2)).reshape(1, H2P)

    w3p = jnp.pad(W3.T, ((0, H2P - n2), (0, OUTP - 1)))
    b3p = jnp.pad(b3, (0, OUTP - 1)).reshape(1, OUTP)

    xemb = _sc_gather(table2d, fidx).reshape(B, EMB_COLS)
    h1, st1 = _tc1(xemb, xc, w1e, w1c, b1p, bncg, bncb)
    h2, st2 = _tc_mid(h1, st1, w2p, b2p, g1p, bb1p)
    o = _tc_last(h2, st2, w3p, b3p, g2p, bb2p)
    return o[:, :1]


# speedup vs baseline: 28.0507x; 3.9889x over previous
import functools

import jax
import jax.numpy as jnp
from jax import lax
from jax.experimental import pallas as pl
from jax.experimental.pallas import tpu as pltpu
from jax.experimental.pallas import tpu_sc as plsc

N_FIELDS = 26
VOCAB = 100000
EMB_DIM = 16
N_CONT = 13
B = 16384
EMB_COLS = N_FIELDS * EMB_DIM
EPS = 1e-5

CH = 128
BLK = 512
H1P = 1024
H2P = 512
CONTP = 16
OUTP = 128


CHB = 8192


def _sc_gather(tableT, xcatT):
    info = plsc.get_sparse_core_info()
    nc, ns = info.num_cores, info.num_subcores
    nw = nc * ns
    npairs = N_FIELDS * EMB_DIM
    per_w = npairs // nw
    assert npairs == per_w * nw
    nchunk = B // CHB
    mesh = plsc.VectorSubcoreMesh(core_axis_name="c", subcore_axis_name="s")

    @functools.partial(
        pl.kernel,
        mesh=mesh,
        out_type=jax.ShapeDtypeStruct((npairs, B), jnp.float32),
        scratch_types=[
            pltpu.VMEM((VOCAB,), jnp.float32),
            pltpu.VMEM((CHB,), jnp.int32),
            pltpu.VMEM((CHB,), jnp.float32),
        ],
        compiler_params=pltpu.CompilerParams(needs_layout_passes=False),
    )
    def gk(table_hbm, idx_hbm, out_hbm, row_v, idx_v, o_v):
        wid = lax.axis_index("s") * nc + lax.axis_index("c")

        def pair_body(p, carry):
            pid = wid * per_w + p
            f = pid // EMB_DIM
            d = pid % EMB_DIM
            pltpu.sync_copy(table_hbm.at[f, d], row_v)

            def chunk_body(cb, carry2):
                pltpu.sync_copy(idx_hbm.at[f, pl.ds(cb * CHB, CHB)], idx_v)

                def vec_body(j, carry3):
                    for u in range(8):
                        o = (j * 8 + u) * 16
                        idx = idx_v[pl.ds(o, 16)]
                        o_v[pl.ds(o, 16)] = plsc.load_gather(row_v, [idx])
                    return carry3

                lax.fori_loop(0, CHB // 128, vec_body, 0)
                pltpu.sync_copy(o_v, out_hbm.at[pid, pl.ds(cb * CHB, CHB)])
                return carry2

            lax.fori_loop(0, nchunk, chunk_body, 0)
            return carry

        lax.fori_loop(0, per_w, pair_body, 0)

    return gk(tableT, xcatT)


def _tc1(xemb, xc, w1e, w1c, b1, bng, bnb):
    nblk = B // BLK

    def body(xe_ref, xc_ref, w1e_ref, w1c_ref, b1_ref, g_ref, bb_ref,
             h1_ref, st_ref, aff_ref):
        i = pl.program_id(0)

        @pl.when(i == 0)
        def _():
            xcf = xc_ref[...]
            m = jnp.mean(xcf, axis=0, keepdims=True)
            v = jnp.mean(xcf * xcf, axis=0, keepdims=True) - m * m
            sc = g_ref[...] / jnp.sqrt(v + EPS)
            aff_ref[0:1, :] = sc
            aff_ref[1:2, :] = bb_ref[...] - m * sc
            st_ref[...] = jnp.zeros_like(st_ref)

        xcb = xc_ref[pl.ds(i * BLK, BLK), :]
        xcn = xcb * aff_ref[0:1, :] + aff_ref[1:2, :]
        h = lax.dot_general(xe_ref[...], w1e_ref[...],
                            (((0,), (0,)), ((), ())),
                            preferred_element_type=jnp.float32)
        h = h + jnp.dot(xcn, w1c_ref[...], preferred_element_type=jnp.float32)
        h = jnp.maximum(h + b1_ref[...], 0.0)
        h1_ref[...] = h
        st_ref[0:1, :] += jnp.sum(h, axis=0, keepdims=True)
        st_ref[1:2, :] += jnp.sum(h * h, axis=0, keepdims=True)

    return pl.pallas_call(
        body,
        grid=(nblk,),
        in_specs=[
            pl.BlockSpec((EMB_COLS, BLK), lambda i: (0, i)),
            pl.BlockSpec((B, CONTP), lambda i: (0, 0)),
            pl.BlockSpec((EMB_COLS, H1P), lambda i: (0, 0)),
            pl.BlockSpec((CONTP, H1P), lambda i: (0, 0)),
            pl.BlockSpec((1, H1P), lambda i: (0, 0)),
            pl.BlockSpec((1, CONTP), lambda i: (0, 0)),
            pl.BlockSpec((1, CONTP), lambda i: (0, 0)),
        ],
        out_specs=[
            pl.BlockSpec((BLK, H1P), lambda i: (i, 0)),
            pl.BlockSpec((8, H1P), lambda i: (0, 0)),
        ],
        out_shape=[
            jax.ShapeDtypeStruct((B, H1P), jnp.float32),
            jax.ShapeDtypeStruct((8, H1P), jnp.float32),
        ],
        scratch_shapes=[pltpu.VMEM((8, CONTP), jnp.float32)],
        compiler_params=pltpu.CompilerParams(
            dimension_semantics=("arbitrary",)),
    )(xemb, xc, w1e, w1c, b1, bng, bnb)


def _tc_mid(h1, st1, w2, b2, g1, bb1):
    nblk = B // BLK

    def body(h1_ref, st_ref_in, w2_ref, b2_ref, g_ref, bb_ref,
             h2_ref, st2_ref, aff_ref):
        i = pl.program_id(0)

        @pl.when(i == 0)
        def _():
            m = st_ref_in[0:1, :] * (1.0 / B)
            v = st_ref_in[1:2, :] * (1.0 / B) - m * m
            sc = g_ref[...] / jnp.sqrt(v + EPS)
            aff_ref[0:1, :] = sc
            aff_ref[1:2, :] = bb_ref[...] - m * sc
            st2_ref[...] = jnp.zeros_like(st2_ref)

        h1n = h1_ref[...] * aff_ref[0:1, :] + aff_ref[1:2, :]
        h = jnp.dot(h1n, w2_ref[...], preferred_element_type=jnp.float32)
        h = jnp.maximum(h + b2_ref[...], 0.0)
        h2_ref[...] = h
        st2_ref[0:1, :] += jnp.sum(h, axis=0, keepdims=True)
        st2_ref[1:2, :] += jnp.sum(h * h, axis=0, keepdims=True)

    return pl.pallas_call(
        body,
        grid=(nblk,),
        in_specs=[
            pl.BlockSpec((BLK, H1P), lambda i: (i, 0)),
            pl.BlockSpec((8, H1P), lambda i: (0, 0)),
            pl.BlockSpec((H1P, H2P), lambda i: (0, 0)),
            pl.BlockSpec((1, H2P), lambda i: (0, 0)),
            pl.BlockSpec((1, H1P), lambda i: (0, 0)),
            pl.BlockSpec((1, H1P), lambda i: (0, 0)),
        ],
        out_specs=[
            pl.BlockSpec((BLK, H2P), lambda i: (i, 0)),
            pl.BlockSpec((8, H2P), lambda i: (0, 0)),
        ],
        out_shape=[
            jax.ShapeDtypeStruct((B, H2P), jnp.float32),
            jax.ShapeDtypeStruct((8, H2P), jnp.float32),
        ],
        scratch_shapes=[pltpu.VMEM((8, H1P), jnp.float32)],
        compiler_params=pltpu.CompilerParams(
            dimension_semantics=("arbitrary",)),
    )(h1, st1, w2, b2, g1, bb1)


def _tc_last(h2, st2, w3, b3, g2, bb2):
    nblk = B // BLK

    def body(h2_ref, st_ref_in, w3_ref, b3_ref, g_ref, bb_ref,
             o_ref, aff_ref):
        i = pl.program_id(0)

        @pl.when(i == 0)
        def _():
            m = st_ref_in[0:1, :] * (1.0 / B)
            v = st_ref_in[1:2, :] * (1.0 / B) - m * m
            sc = g_ref[...] / jnp.sqrt(v + EPS)
            aff_ref[0:1, :] = sc
            aff_ref[1:2, :] = bb_ref[...] - m * sc

        h2n = h2_ref[...] * aff_ref[0:1, :] + aff_ref[1:2, :]
        o = jnp.dot(h2n, w3_ref[...], preferred_element_type=jnp.float32)
        o_ref[...] = o + b3_ref[...]

    return pl.pallas_call(
        body,
        grid=(nblk,),
        in_specs=[
            pl.BlockSpec((BLK, H2P), lambda i: (i, 0)),
            pl.BlockSpec((8, H2P), lambda i: (0, 0)),
            pl.BlockSpec((H2P, OUTP), lambda i: (0, 0)),
            pl.BlockSpec((1, OUTP), lambda i: (0, 0)),
            pl.BlockSpec((1, H2P), lambda i: (0, 0)),
            pl.BlockSpec((1, H2P), lambda i: (0, 0)),
        ],
        out_specs=pl.BlockSpec((BLK, OUTP), lambda i: (i, 0)),
        out_shape=jax.ShapeDtypeStruct((B, OUTP), jnp.float32),
        scratch_shapes=[pltpu.VMEM((8, H2P), jnp.float32)],
        compiler_params=pltpu.CompilerParams(
            dimension_semantics=("arbitrary",)),
    )(h2, st2, w3, b3, g2, bb2)


def kernel(x_cat, x_cont, emb_tables, bn_cont_g, bn_cont_b,
           W1, b1, bn1_g, bn1_b, W2, b2, bn2_g, bn2_b, W3, b3):
    tableT = jnp.transpose(emb_tables, (0, 2, 1))
    xcatT = jnp.transpose(x_cat.astype(jnp.int32), (1, 0))

    xc = jnp.pad(x_cont, ((0, 0), (0, CONTP - N_CONT)))
    bncg = jnp.pad(bn_cont_g, (0, CONTP - N_CONT)).reshape(1, CONTP)
    bncb = jnp.pad(bn_cont_b, (0, CONTP - N_CONT)).reshape(1, CONTP)

    n1 = W1.shape[0]
    n2 = W2.shape[0]
    w1e = jnp.pad(W1[:, :EMB_COLS].T, ((0, 0), (0, H1P - n1)))
    w1c = jnp.pad(W1[:, EMB_COLS:].T,
                  ((0, CONTP - N_CONT), (0, H1P - n1)))
    b1p = jnp.pad(b1, (0, H1P - n1)).reshape(1, H1P)
    g1p = jnp.pad(bn1_g, (0, H1P - n1)).reshape(1, H1P)
    bb1p = jnp.pad(bn1_b, (0, H1P - n1)).reshape(1, H1P)

    w2p = jnp.pad(W2.T, ((0, H1P - n1), (0, H2P - n2)))
    b2p = jnp.pad(b2, (0, H2P - n2)).reshape(1, H2P)
    g2p = jnp.pad(bn2_g, (0, H2P - n2)).reshape(1, H2P)
    bb2p = jnp.pad(bn2_b, (0, H2P - n2)).reshape(1, H2P)

    w3p = jnp.pad(W3.T, ((0, H2P - n2), (0, OUTP - 1)))
    b3p = jnp.pad(b3, (0, OUTP - 1)).reshape(1, OUTP)

    xembT = _sc_gather(tableT, xcatT)
    h1, st1 = _tc1(xembT, xc, w1e, w1c, b1p, bncg, bncb)
    h2, st2 = _tc_mid(h1, st1, w2p, b2p, g1p, bb1p)
    o = _tc_last(h2, st2, w3p, b3p, g2p, bb2p)
    return o[:, :1]


# speedup vs baseline: 31.9314x; 1.1383x over previous
import functools

import jax
import jax.numpy as jnp
from jax import lax
from jax.experimental import pallas as pl
from jax.experimental.pallas import tpu as pltpu
from jax.experimental.pallas import tpu_sc as plsc

N_FIELDS = 26
VOCAB = 100000
EMB_DIM = 16
N_CONT = 13
B = 16384
EMB_COLS = N_FIELDS * EMB_DIM
EPS = 1e-5

CH = 128
BLK = 512
H1P = 1024
H2P = 512
CONTP = 16
OUTP = 128


CHB = 8192


def _sc_gather(tableT, xcatT):
    info = plsc.get_sparse_core_info()
    nc, ns = info.num_cores, info.num_subcores
    nw = nc * ns
    npairs = N_FIELDS * EMB_DIM
    per_w = npairs // nw
    assert npairs == per_w * nw
    nchunk = B // CHB
    mesh = plsc.VectorSubcoreMesh(core_axis_name="c", subcore_axis_name="s")

    @functools.partial(
        pl.kernel,
        mesh=mesh,
        out_type=jax.ShapeDtypeStruct((npairs, B), jnp.float32),
        scratch_types=[
            pltpu.VMEM((VOCAB,), jnp.float32),
            pltpu.VMEM((CHB,), jnp.int32),
            pltpu.VMEM((CHB,), jnp.float32),
        ],
        compiler_params=pltpu.CompilerParams(needs_layout_passes=False),
    )
    def gk(table_hbm, idx_hbm, out_hbm, row_v, idx_v, o_v):
        wid = lax.axis_index("s") * nc + lax.axis_index("c")

        def pair_body(p, carry):
            pid = wid * per_w + p
            f = pid // EMB_DIM
            d = pid % EMB_DIM
            pltpu.sync_copy(table_hbm.at[f, d], row_v)

            def chunk_body(cb, carry2):
                pltpu.sync_copy(idx_hbm.at[f, pl.ds(cb * CHB, CHB)], idx_v)

                def vec_body(j, carry3):
                    for u in range(8):
                        o = (j * 8 + u) * 16
                        idx = idx_v[pl.ds(o, 16)]
                        o_v[pl.ds(o, 16)] = plsc.load_gather(row_v, [idx])
                    return carry3

                lax.fori_loop(0, CHB // 128, vec_body, 0)
                pltpu.sync_copy(o_v, out_hbm.at[pid, pl.ds(cb * CHB, CHB)])
                return carry2

            lax.fori_loop(0, nchunk, chunk_body, 0)
            return carry

        lax.fori_loop(0, per_w, pair_body, 0)

    return gk(tableT, xcatT)


NBLK = B // BLK


def _tc_mlp(xembT, xc, w1e, w1c, b1, bncg, bncb, w2, b2, g1, bb1,
            w3, b3, g2, bb2):

    def body(xt_ref, xc_ref, w1e_ref, w1c_ref, b1_ref, bncg_ref, bncb_ref,
             w2_ref, b2_ref, g1_ref, bb1_ref, w3_ref, b3_ref, g2_ref,
             bb2_ref, o_ref, h1_s, h2_s, st1, st2, affc, aff1, aff2):
        p = pl.program_id(0)
        i = pl.program_id(1)

        @pl.when((p == 0) & (i == 0))
        def _():
            xcf = xc_ref[...]
            m = jnp.mean(xcf, axis=0, keepdims=True)
            v = jnp.mean(xcf * xcf, axis=0, keepdims=True) - m * m
            sc = bncg_ref[...] / jnp.sqrt(v + EPS)
            affc[0:1, :] = sc
            affc[1:2, :] = bncb_ref[...] - m * sc
            st1[...] = jnp.zeros_like(st1)
            st2[...] = jnp.zeros_like(st2)

        @pl.when(p == 0)
        def _():
            xcb = xc_ref[pl.ds(i * BLK, BLK), :]
            xcn = xcb * affc[0:1, :] + affc[1:2, :]
            h = lax.dot_general(xt_ref[...], w1e_ref[...],
                                (((0,), (0,)), ((), ())),
                                preferred_element_type=jnp.float32)
            h = h + jnp.dot(xcn, w1c_ref[...],
                            preferred_element_type=jnp.float32)
            h = jnp.maximum(h + b1_ref[...], 0.0)
            h1_s[pl.ds(i * BLK, BLK), :] = h.astype(jnp.bfloat16)
            st1[0:1, :] += jnp.sum(h, axis=0, keepdims=True)
            st1[1:2, :] += jnp.sum(h * h, axis=0, keepdims=True)

        @pl.when((p == 1) & (i == 0))
        def _():
            m = st1[0:1, :] * (1.0 / B)
            v = st1[1:2, :] * (1.0 / B) - m * m
            sc = g1_ref[...] / jnp.sqrt(v + EPS)
            aff1[0:1, :] = sc
            aff1[1:2, :] = bb1_ref[...] - m * sc

        @pl.when(p == 1)
        def _():
            h1b = h1_s[pl.ds(i * BLK, BLK), :].astype(jnp.float32)
            h1n = h1b * aff1[0:1, :] + aff1[1:2, :]
            h = jnp.dot(h1n, w2_ref[...], preferred_element_type=jnp.float32)
            h = jnp.maximum(h + b2_ref[...], 0.0)
            h2_s[pl.ds(i * BLK, BLK), :] = h.astype(jnp.bfloat16)
            st2[0:1, :] += jnp.sum(h, axis=0, keepdims=True)
            st2[1:2, :] += jnp.sum(h * h, axis=0, keepdims=True)

        @pl.when((p == 2) & (i == 0))
        def _():
            m = st2[0:1, :] * (1.0 / B)
            v = st2[1:2, :] * (1.0 / B) - m * m
            sc = g2_ref[...] / jnp.sqrt(v + EPS)
            aff2[0:1, :] = sc
            aff2[1:2, :] = bb2_ref[...] - m * sc

        @pl.when(p == 2)
        def _():
            h2b = h2_s[pl.ds(i * BLK, BLK), :].astype(jnp.float32)
            h2n = h2b * aff2[0:1, :] + aff2[1:2, :]
            o = jnp.dot(h2n, w3_ref[...], preferred_element_type=jnp.float32)
            o_ref[...] = o + b3_ref[...]

    cnst = lambda p, i: (0, 0)
    return pl.pallas_call(
        body,
        grid=(3, NBLK),
        in_specs=[
            pl.BlockSpec((EMB_COLS, BLK),
                         lambda p, i: (0, jnp.where(p == 0, i, NBLK - 1))),
            pl.BlockSpec((B, CONTP), cnst),
            pl.BlockSpec((EMB_COLS, H1P), cnst),
            pl.BlockSpec((CONTP, H1P), cnst),
            pl.BlockSpec((1, H1P), cnst),
            pl.BlockSpec((1, CONTP), cnst),
            pl.BlockSpec((1, CONTP), cnst),
            pl.BlockSpec((H1P, H2P), cnst),
            pl.BlockSpec((1, H2P), cnst),
            pl.BlockSpec((1, H1P), cnst),
            pl.BlockSpec((1, H1P), cnst),
            pl.BlockSpec((H2P, OUTP), cnst),
            pl.BlockSpec((1, OUTP), cnst),
            pl.BlockSpec((1, H2P), cnst),
            pl.BlockSpec((1, H2P), cnst),
        ],
        out_specs=pl.BlockSpec((BLK, OUTP),
                               lambda p, i: (jnp.where(p == 2, i, 0), 0)),
        out_shape=jax.ShapeDtypeStruct((B, OUTP), jnp.float32),
        scratch_shapes=[
            pltpu.VMEM((B, H1P), jnp.bfloat16),
            pltpu.VMEM((B, H2P), jnp.bfloat16),
            pltpu.VMEM((8, H1P), jnp.float32),
            pltpu.VMEM((8, H2P), jnp.float32),
            pltpu.VMEM((8, CONTP), jnp.float32),
            pltpu.VMEM((8, H1P), jnp.float32),
            pltpu.VMEM((8, H2P), jnp.float32),
        ],
        compiler_params=pltpu.CompilerParams(
            dimension_semantics=("arbitrary", "arbitrary"),
            vmem_limit_bytes=64 * 1024 * 1024),
    )(xembT, xc, w1e, w1c, b1, bncg, bncb, w2, b2, g1, bb1, w3, b3, g2, bb2)


def kernel(x_cat, x_cont, emb_tables, bn_cont_g, bn_cont_b,
           W1, b1, bn1_g, bn1_b, W2, b2, bn2_g, bn2_b, W3, b3):
    tableT = jnp.transpose(emb_tables, (0, 2, 1))
    xcatT = jnp.transpose(x_cat.astype(jnp.int32), (1, 0))

    xc = jnp.pad(x_cont, ((0, 0), (0, CONTP - N_CONT)))
    bncg = jnp.pad(bn_cont_g, (0, CONTP - N_CONT)).reshape(1, CONTP)
    bncb = jnp.pad(bn_cont_b, (0, CONTP - N_CONT)).reshape(1, CONTP)

    n1 = W1.shape[0]
    n2 = W2.shape[0]
    w1e = jnp.pad(W1[:, :EMB_COLS].T, ((0, 0), (0, H1P - n1)))
    w1c = jnp.pad(W1[:, EMB_COLS:].T,
                  ((0, CONTP - N_CONT), (0, H1P - n1)))
    b1p = jnp.pad(b1, (0, H1P - n1)).reshape(1, H1P)
    g1p = jnp.pad(bn1_g, (0, H1P - n1)).reshape(1, H1P)
    bb1p = jnp.pad(bn1_b, (0, H1P - n1)).reshape(1, H1P)

    w2p = jnp.pad(W2.T, ((0, H1P - n1), (0, H2P - n2)))
    b2p = jnp.pad(b2, (0, H2P - n2)).reshape(1, H2P)
    g2p = jnp.pad(bn2_g, (0, H2P - n2)).reshape(1, H2P)
    bb2p = jnp.pad(bn2_b, (0, H2P - n2)).reshape(1, H2P)

    w3p = jnp.pad(W3.T, ((0, H2P - n2), (0, OUTP - 1)))
    b3p = jnp.pad(b3, (0, OUTP - 1)).reshape(1, OUTP)

    xembT = _sc_gather(tableT, xcatT)
    o = _tc_mlp(xembT, xc, w1e, w1c, b1p, bncg, bncb,
                w2p, b2p, g1p, bb1p, w3p, b3p, g2p, bb2p)
    return o[:, :1]


# speedup vs baseline: 37.8557x; 1.1855x over previous
import functools

import jax
import jax.numpy as jnp
from jax import lax
from jax.experimental import pallas as pl
from jax.experimental.pallas import tpu as pltpu
from jax.experimental.pallas import tpu_sc as plsc

N_FIELDS = 26
VOCAB = 100000
EMB_DIM = 16
N_CONT = 13
B = 16384
EMB_COLS = N_FIELDS * EMB_DIM
EPS = 1e-5

CH = 128
BLK = 512
H1P = 1024
H2P = 512
CONTP = 16
OUTP = 128


CHB = 8192


def _sc_gather(tableT, xcatT):
    info = plsc.get_sparse_core_info()
    nc, ns = info.num_cores, info.num_subcores
    nw = nc * ns
    npairs = N_FIELDS * EMB_DIM
    per_w = npairs // nw
    assert npairs == per_w * nw
    nchunk = B // CHB
    mesh = plsc.VectorSubcoreMesh(core_axis_name="c", subcore_axis_name="s")

    @functools.partial(
        pl.kernel,
        mesh=mesh,
        out_type=jax.ShapeDtypeStruct((npairs, B), jnp.float32),
        scratch_types=[
            pltpu.VMEM((VOCAB,), jnp.float32),
            pltpu.VMEM((CHB,), jnp.int32),
            pltpu.VMEM((CHB,), jnp.float32),
            pltpu.VMEM((CHB,), jnp.float32),
            pltpu.SemaphoreType.DMA,
            pltpu.SemaphoreType.DMA,
            pltpu.SemaphoreType.DMA,
        ],
        compiler_params=pltpu.CompilerParams(needs_layout_passes=False),
    )
    def gk(table_hbm, idx_hbm, out_hbm, row_v, idx_v, o_v0, o_v1,
           rsem, s0, s1):
        wid = lax.axis_index("s") * nc + lax.axis_index("c")
        o_bufs = (o_v0, o_v1)
        o_sems = (s0, s1)

        def pair_body(p, carry):
            pid = wid * per_w + p
            f = pid // EMB_DIM
            d = pid % EMB_DIM
            rcp = pltpu.async_copy(table_hbm.at[f, d], row_v, rsem)
            rcp.wait()

            for cb in range(nchunk):
                o_v = o_bufs[cb % 2]
                pltpu.sync_copy(idx_hbm.at[f, pl.ds(cb * CHB, CHB)], idx_v)

                @pl.when(p > 0)
                def _():
                    pltpu.make_async_copy(
                        o_v, out_hbm.at[pid, pl.ds(cb * CHB, CHB)],
                        o_sems[cb % 2]).wait()

                @plsc.parallel_loop(0, CHB // 16, unroll=8)
                def _(j):
                    o = j * 16
                    idx = idx_v[pl.ds(o, 16)]
                    o_v[pl.ds(o, 16)] = plsc.load_gather(row_v, [idx])

                pltpu.async_copy(
                    o_v, out_hbm.at[pid, pl.ds(cb * CHB, CHB)],
                    o_sems[cb % 2])
            return carry

        lax.fori_loop(0, per_w, pair_body, 0)
        pltpu.make_async_copy(o_v0, out_hbm.at[0, pl.ds(0, CHB)], s0).wait()
        pltpu.make_async_copy(o_v1, out_hbm.at[0, pl.ds(CHB, CHB)], s1).wait()

    return gk(tableT, xcatT)


NBLK = B // BLK


def _tc_mlp(xembT, xc, w1e, w1c, b1, bncg, bncb, w2, b2, g1, bb1,
            w3, b3, g2, bb2):

    def body(xt_ref, xc_ref, w1e_ref, w1c_ref, b1_ref, bncg_ref, bncb_ref,
             w2_ref, b2_ref, g1_ref, bb1_ref, w3_ref, b3_ref, g2_ref,
             bb2_ref, o_ref, h1_s, h2_s, st1, st2, affc, aff1, aff2):
        p = pl.program_id(0)
        i = pl.program_id(1)

        @pl.when((p == 0) & (i == 0))
        def _():
            xcf = xc_ref[...]
            m = jnp.mean(xcf, axis=0, keepdims=True)
            v = jnp.mean(xcf * xcf, axis=0, keepdims=True) - m * m
            sc = bncg_ref[...] / jnp.sqrt(v + EPS)
            affc[0:1, :] = sc
            affc[1:2, :] = bncb_ref[...] - m * sc
            st1[...] = jnp.zeros_like(st1)
            st2[...] = jnp.zeros_like(st2)

        @pl.when(p == 0)
        def _():
            xcb = xc_ref[pl.ds(i * BLK, BLK), :]
            xcn = xcb * affc[0:1, :] + affc[1:2, :]
            h = lax.dot_general(xt_ref[...], w1e_ref[...],
                                (((0,), (0,)), ((), ())),
                                preferred_element_type=jnp.float32)
            h = h + jnp.dot(xcn, w1c_ref[...],
                            preferred_element_type=jnp.float32)
            h = jnp.maximum(h + b1_ref[...], 0.0)
            h1_s[pl.ds(i * BLK, BLK), :] = h.astype(jnp.bfloat16)
            st1[0:1, :] += jnp.sum(h, axis=0, keepdims=True)
            st1[1:2, :] += jnp.sum(h * h, axis=0, keepdims=True)

        @pl.when((p == 1) & (i == 0))
        def _():
            m = st1[0:1, :] * (1.0 / B)
            v = st1[1:2, :] * (1.0 / B) - m * m
            sc = g1_ref[...] / jnp.sqrt(v + EPS)
            aff1[0:1, :] = sc
            aff1[1:2, :] = bb1_ref[...] - m * sc

        @pl.when(p == 1)
        def _():
            h1b = h1_s[pl.ds(i * BLK, BLK), :].astype(jnp.float32)
            h1n = h1b * aff1[0:1, :] + aff1[1:2, :]
            h = jnp.dot(h1n, w2_ref[...], preferred_element_type=jnp.float32)
            h = jnp.maximum(h + b2_ref[...], 0.0)
            h2_s[pl.ds(i * BLK, BLK), :] = h.astype(jnp.bfloat16)
            st2[0:1, :] += jnp.sum(h, axis=0, keepdims=True)
            st2[1:2, :] += jnp.sum(h * h, axis=0, keepdims=True)

        @pl.when((p == 2) & (i == 0))
        def _():
            m = st2[0:1, :] * (1.0 / B)
            v = st2[1:2, :] * (1.0 / B) - m * m
            sc = g2_ref[...] / jnp.sqrt(v + EPS)
            aff2[0:1, :] = sc
            aff2[1:2, :] = bb2_ref[...] - m * sc

        @pl.when(p == 2)
        def _():
            h2b = h2_s[pl.ds(i * BLK, BLK), :].astype(jnp.float32)
            h2n = h2b * aff2[0:1, :] + aff2[1:2, :]
            o = jnp.dot(h2n, w3_ref[...], preferred_element_type=jnp.float32)
            o_ref[...] = o + b3_ref[...]

    cnst = lambda p, i: (0, 0)
    return pl.pallas_call(
        body,
        grid=(3, NBLK),
        in_specs=[
            pl.BlockSpec((EMB_COLS, BLK),
                         lambda p, i: (0, jnp.where(p == 0, i, NBLK - 1))),
            pl.BlockSpec((B, CONTP), cnst),
            pl.BlockSpec((EMB_COLS, H1P), cnst),
            pl.BlockSpec((CONTP, H1P), cnst),
            pl.BlockSpec((1, H1P), cnst),
            pl.BlockSpec((1, CONTP), cnst),
            pl.BlockSpec((1, CONTP), cnst),
            pl.BlockSpec((H1P, H2P), cnst),
            pl.BlockSpec((1, H2P), cnst),
            pl.BlockSpec((1, H1P), cnst),
            pl.BlockSpec((1, H1P), cnst),
            pl.BlockSpec((H2P, OUTP), cnst),
            pl.BlockSpec((1, OUTP), cnst),
            pl.BlockSpec((1, H2P), cnst),
            pl.BlockSpec((1, H2P), cnst),
        ],
        out_specs=pl.BlockSpec((BLK, OUTP),
                               lambda p, i: (jnp.where(p == 2, i, 0), 0)),
        out_shape=jax.ShapeDtypeStruct((B, OUTP), jnp.float32),
        scratch_shapes=[
            pltpu.VMEM((B, H1P), jnp.bfloat16),
            pltpu.VMEM((B, H2P), jnp.bfloat16),
            pltpu.VMEM((8, H1P), jnp.float32),
            pltpu.VMEM((8, H2P), jnp.float32),
            pltpu.VMEM((8, CONTP), jnp.float32),
            pltpu.VMEM((8, H1P), jnp.float32),
            pltpu.VMEM((8, H2P), jnp.float32),
        ],
        compiler_params=pltpu.CompilerParams(
            dimension_semantics=("arbitrary", "arbitrary"),
            vmem_limit_bytes=64 * 1024 * 1024),
    )(xembT, xc, w1e, w1c, b1, bncg, bncb, w2, b2, g1, bb1, w3, b3, g2, bb2)


def kernel(x_cat, x_cont, emb_tables, bn_cont_g, bn_cont_b,
           W1, b1, bn1_g, bn1_b, W2, b2, bn2_g, bn2_b, W3, b3):
    tableT = jnp.transpose(emb_tables, (0, 2, 1))
    xcatT = jnp.transpose(x_cat.astype(jnp.int32), (1, 0))

    xc = jnp.pad(x_cont, ((0, 0), (0, CONTP - N_CONT)))
    bncg = jnp.pad(bn_cont_g, (0, CONTP - N_CONT)).reshape(1, CONTP)
    bncb = jnp.pad(bn_cont_b, (0, CONTP - N_CONT)).reshape(1, CONTP)

    n1 = W1.shape[0]
    n2 = W2.shape[0]
    w1e = jnp.pad(W1[:, :EMB_COLS].T, ((0, 0), (0, H1P - n1)))
    w1c = jnp.pad(W1[:, EMB_COLS:].T,
                  ((0, CONTP - N_CONT), (0, H1P - n1)))
    b1p = jnp.pad(b1, (0, H1P - n1)).reshape(1, H1P)
    g1p = jnp.pad(bn1_g, (0, H1P - n1)).reshape(1, H1P)
    bb1p = jnp.pad(bn1_b, (0, H1P - n1)).reshape(1, H1P)

    w2p = jnp.pad(W2.T, ((0, H1P - n1), (0, H2P - n2)))
    b2p = jnp.pad(b2, (0, H2P - n2)).reshape(1, H2P)
    g2p = jnp.pad(bn2_g, (0, H2P - n2)).reshape(1, H2P)
    bb2p = jnp.pad(bn2_b, (0, H2P - n2)).reshape(1, H2P)

    w3p = jnp.pad(W3.T, ((0, H2P - n2), (0, OUTP - 1)))
    b3p = jnp.pad(b3, (0, OUTP - 1)).reshape(1, OUTP)

    xembT = _sc_gather(tableT, xcatT)
    o = _tc_mlp(xembT, xc, w1e, w1c, b1p, bncg, bncb,
                w2p, b2p, g1p, bb1p, w3p, b3p, g2p, bb2p)
    return o[:, :1]
